# baseline (speedup 1.0000x reference)
import functools

import jax
import jax.numpy as jnp
from jax import lax
from jax.experimental import pallas as pl
from jax.experimental.pallas import tpu as pltpu
from jax.experimental.pallas import tpu_sc as plsc

_NUM_CORES = 2
_NUM_SUBCORES = 16
_NUM_WORKERS = _NUM_CORES * _NUM_SUBCORES
_LANES = 16


@functools.lru_cache(maxsize=None)
def _make_gather_pool(B, CTX, D):
    bpw = B // _NUM_WORKERS
    ipw = bpw * CTX
    mesh = plsc.VectorSubcoreMesh(core_axis_name="c", subcore_axis_name="s")

    @functools.partial(
        pl.kernel,
        mesh=mesh,
        out_type=jax.ShapeDtypeStruct((B, D), jnp.float32),
        scratch_types=[
            pltpu.VMEM((ipw,), jnp.int32),
            pltpu.VMEM((ipw, D), jnp.float32),
            pltpu.VMEM((bpw, D), jnp.float32),
            pltpu.SemaphoreType.DMA,
        ],
        compiler_params=pltpu.CompilerParams(use_tc_tiling_on_sc=False),
    )
    def gather_pool(ids_hbm, table_hbm, out_hbm, idx_v, rows_v, pooled_v, sem):
        wid = lax.axis_index("s") * _NUM_CORES + lax.axis_index("c")
        base = wid * ipw
        pltpu.sync_copy(ids_hbm.at[pl.ds(base, ipw)], idx_v)
        pltpu.async_copy(table_hbm.at[idx_v], rows_v, sem).wait()
        scale = jnp.float32(1.0 / CTX)

        def body(b, carry):
            for d in range(D // _LANES):
                sl = pl.ds(d * _LANES, _LANES)
                acc = rows_v[b * CTX, sl]
                for c in range(1, CTX):
                    acc = acc + rows_v[b * CTX + c, sl]
                pooled_v[b, sl] = acc * scale
            return carry

        lax.fori_loop(0, bpw, body, 0)
        pltpu.sync_copy(pooled_v, out_hbm.at[pl.ds(wid * bpw, bpw)])

    return gather_pool


@functools.lru_cache(maxsize=None)
def _make_project(B, D, V, TV, NBUF):
    grid_n = pl.cdiv(V, TV)
    rem = V - (grid_n - 1) * TV

    def _out_copy(o_hbm, acc_ref, sems, j, slot, width):
        return pltpu.make_async_copy(
            acc_ref.at[slot, pl.ds(0, width), :],
            o_hbm.at[pl.ds(j * TV, width), :],
            sems.at[slot],
        )

    def body(x_ref, w_ref, o_hbm, acc_ref, sems):
        i = pl.program_id(0)
        slot = lax.rem(i, NBUF)

        @pl.when(i >= NBUF)
        def _():
            _out_copy(o_hbm, acc_ref, sems, i - NBUF, slot, TV).wait()

        acc_ref[slot] = lax.dot_general(
            w_ref[...], x_ref[...],
            dimension_numbers=(((0,), (1,)), ((), ())),
            preferred_element_type=jnp.float32,
        )

        @pl.when(i < grid_n - 1)
        def _():
            _out_copy(o_hbm, acc_ref, sems, i, slot, TV).start()

        @pl.when(i == grid_n - 1)
        def _():
            _out_copy(o_hbm, acc_ref, sems, i, slot, rem).start()
            for j in range(max(0, grid_n - NBUF), grid_n):
                w_j = TV if j < grid_n - 1 else rem
                _out_copy(o_hbm, acc_ref, sems, j, j % NBUF, w_j).wait()

    return pl.pallas_call(
        body,
        grid=(grid_n,),
        in_specs=[
            pl.BlockSpec((B, D), lambda i: (0, 0)),
            pl.BlockSpec((D, TV), lambda i: (0, i)),
        ],
        out_specs=pl.BlockSpec(memory_space=pl.ANY),
        out_shape=jax.ShapeDtypeStruct((V, B), jnp.float32),
        scratch_shapes=[
            pltpu.VMEM((NBUF, TV, B), jnp.float32),
            pltpu.SemaphoreType.DMA((NBUF,)),
        ],
        compiler_params=pltpu.CompilerParams(
            dimension_semantics=("arbitrary",),
        ),
    )


def kernel(context_ids, emb_table, proj_weight):
    B, CTX = context_ids.shape
    V, D = emb_table.shape
    ids = context_ids.reshape(-1).astype(jnp.int32)
    pooled = _make_gather_pool(B, CTX, D)(ids, emb_table)
    out_t = _make_project(B, D, V, 3072, 4)(pooled, proj_weight.T)
    return out_t.T
